# Initial kernel scaffold; baseline (speedup 1.0000x reference)
import functools

import jax
import jax.numpy as jnp
from jax import lax
from jax.experimental import pallas as pl
from jax.experimental.pallas import tpu as pltpu
from jax.experimental.pallas import tpu_sc as plsc

VOCAB = 100000
D = 64
B = 1024
C = 20
NNEG = 20
PAD = 0

K = C + C * NNEG
KP = 432
NW = 32
BPW = B // NW
GROUPS = KP // 16
CHUNKS = ((0, 128), (128, 128), (256, 128), (384, KP - 384))

_mesh = plsc.VectorSubcoreMesh(core_axis_name="c", subcore_axis_name="s")


@functools.partial(
    pl.kernel,
    out_type=jax.ShapeDtypeStruct((B, KP), jnp.float32),
    mesh=_mesh,
    scratch_types=[
        pltpu.VMEM((BPW,), jnp.int32),
        pltpu.VMEM((BPW, D), jnp.float32),
        pltpu.VMEM((KP,), jnp.int32),
        pltpu.VMEM((KP, D), jnp.float32),
        pltpu.VMEM((KP,), jnp.float32),
        pltpu.SemaphoreType.DMA,
    ],
)
def _sc_scores(iword_hbm, okidx_hbm, ovec_hbm, ivec_hbm, out_hbm,
               iw_v, iv_v, idx_v, rows_v, sc_v, sem):
    wid = lax.axis_index("s") * 2 + lax.axis_index("c")
    base_b = wid * BPW
    pltpu.sync_copy(iword_hbm.at[pl.ds(base_b, BPW)], iw_v)
    pltpu.async_copy(ivec_hbm.at[iw_v], iv_v, sem).wait()

    def b_body(bl, carry):
        b = base_b + bl
        pltpu.sync_copy(okidx_hbm.at[b], idx_v)
        cps = [
            pltpu.async_copy(ovec_hbm.at[idx_v.at[pl.ds(st, n)]],
                             rows_v.at[pl.ds(st, n)], sem)
            for st, n in CHUNKS
        ]
        i0 = iv_v[bl, pl.ds(0, 16)]
        i1 = iv_v[bl, pl.ds(16, 16)]
        i2 = iv_v[bl, pl.ds(32, 16)]
        i3 = iv_v[bl, pl.ds(48, 16)]
        for cp in cps:
            cp.wait()
        lane = lax.iota(jnp.int32, 16)

        def g_body(g, carry2):
            acc = jnp.zeros((16,), jnp.float32)
            row0 = pl.multiple_of(g * 16, 16)
            for r in range(16):
                row = row0 + r
                p = (rows_v[row, pl.ds(0, 16)] * i0
                     + rows_v[row, pl.ds(16, 16)] * i1
                     + rows_v[row, pl.ds(32, 16)] * i2
                     + rows_v[row, pl.ds(48, 16)] * i3)
                s = jnp.sum(p)
                acc = jnp.where(lane == r, s, acc)
            sc_v[pl.ds(row0, 16)] = acc
            return carry2

        lax.fori_loop(0, GROUPS, g_body, 0)
        pltpu.sync_copy(sc_v, out_hbm.at[b])
        return carry

    lax.fori_loop(0, BPW, b_body, 0)


def _tc_loss_body(scores_ref, ow_ref, out_ref):
    s = scores_ref[...]
    ow = ow_ref[...]

    def log_sigmoid(x):
        return jnp.minimum(x, 0.0) - jnp.log1p(jnp.exp(-jnp.abs(x)))

    o_sc = s[:, :C]
    n_raw = s[:, C:C + C * NNEG]
    non_pad = (ow != PAD).astype(jnp.float32)
    n_valid = jnp.sum(non_pad)
    oloss = jnp.sum(log_sigmoid(o_sc) * non_pad) / n_valid
    nterm = jnp.sum(log_sigmoid(-n_raw)) / (C * B)
    out_ref[0, 0] = -(oloss + nterm)


def _tc_loss(scores, owords):
    return pl.pallas_call(
        _tc_loss_body,
        out_shape=jax.ShapeDtypeStruct((1, 1), jnp.float32),
        in_specs=[
            pl.BlockSpec(memory_space=pltpu.VMEM),
            pl.BlockSpec(memory_space=pltpu.VMEM),
        ],
        out_specs=pl.BlockSpec(memory_space=pltpu.SMEM),
    )(scores, owords)


def kernel(iword, owords, nwords, ivec_table, ovec_table):
    pad = jnp.zeros((B, KP - K), jnp.int32)
    okidx = jnp.concatenate([owords, nwords, pad], axis=1)
    scores = _sc_scores(iword, okidx, ovec_table, ivec_table)
    loss = _tc_loss(scores, owords)
    return loss[0, 0]


# speedup vs baseline: 3.8500x; 3.8500x over previous
import functools

import jax
import jax.numpy as jnp
import numpy as np
from jax import lax
from jax.experimental import pallas as pl
from jax.experimental.pallas import tpu as pltpu
from jax.experimental.pallas import tpu_sc as plsc

VOCAB = 100000
D = 64
B = 1024
C = 20
NNEG = 20
PAD = 0

K = C + C * NNEG
KP = 432
NW = 32
BPW = B // NW
GROUPS = KP // 16
CHUNKS = ((0, 128), (128, 128), (256, 128), (384, KP - 384))

_mesh = plsc.VectorSubcoreMesh(core_axis_name="c", subcore_axis_name="s")

_GDN = lax.GatherDimensionNumbers(
    offset_dims=(), collapsed_slice_dims=(0,), start_index_map=(0,))


def _take16(v, idx):
    return lax.gather(v, idx.reshape(16, 1), _GDN, (1,),
                      mode=lax.GatherScatterMode.PROMISE_IN_BOUNDS)


def _hsum_bcast(p, perms):
    for perm in perms:
        p = p + _take16(p, perm)
    return p


@functools.partial(
    pl.kernel,
    out_type=jax.ShapeDtypeStruct((B, KP), jnp.float32),
    mesh=_mesh,
    scratch_types=[
        pltpu.VMEM((BPW,), jnp.int32),
        pltpu.VMEM((BPW, D), jnp.float32),
        pltpu.VMEM((KP,), jnp.int32),
        pltpu.VMEM((KP, D), jnp.float32),
        pltpu.VMEM((KP,), jnp.float32),
        pltpu.SemaphoreType.DMA,
    ],
    compiler_params=pltpu.CompilerParams(use_tc_tiling_on_sc=False),
)
def _sc_scores(iword_hbm, okidx_hbm, ovec_hbm, ivec_hbm, out_hbm,
               iw_v, iv_v, idx_v, rows_v, sc_v, sem):
    wid = lax.axis_index("s") * 2 + lax.axis_index("c")
    base_b = wid * BPW
    pltpu.sync_copy(iword_hbm.at[pl.ds(base_b, BPW)], iw_v)
    pltpu.async_copy(ivec_hbm.at[iw_v], iv_v, sem).wait()

    def b_body(bl, carry):
        b = base_b + bl
        pltpu.sync_copy(okidx_hbm.at[b], idx_v)
        cps = [
            pltpu.async_copy(ovec_hbm.at[idx_v.at[pl.ds(st, n)]],
                             rows_v.at[pl.ds(st, n)], sem)
            for st, n in CHUNKS
        ]
        i0 = iv_v[bl, pl.ds(0, 16)]
        i1 = iv_v[bl, pl.ds(16, 16)]
        i2 = iv_v[bl, pl.ds(32, 16)]
        i3 = iv_v[bl, pl.ds(48, 16)]
        for cp in cps:
            cp.wait()
        lane = lax.iota(jnp.int32, 16)
        perms = [lane ^ (1 << k) for k in (3, 2, 1, 0)]

        def g_body(g, carry2):
            acc = jnp.zeros((16,), jnp.float32)
            row0 = pl.multiple_of(g * 16, 16)
            for r in range(16):
                row = row0 + r
                p = (rows_v[row, pl.ds(0, 16)] * i0
                     + rows_v[row, pl.ds(16, 16)] * i1
                     + rows_v[row, pl.ds(32, 16)] * i2
                     + rows_v[row, pl.ds(48, 16)] * i3)
                acc = jnp.where(lane == r, _hsum_bcast(p, perms), acc)
            sc_v[pl.ds(row0, 16)] = acc
            return carry2

        lax.fori_loop(0, GROUPS, g_body, 0)
        pltpu.sync_copy(sc_v, out_hbm.at[b])
        return carry

    lax.fori_loop(0, BPW, b_body, 0)


def _tc_loss_body(scores_ref, ow_ref, out_ref):
    s = scores_ref[...]
    ow = ow_ref[...]

    def log_sigmoid(x):
        return jnp.minimum(x, 0.0) - jnp.log1p(jnp.exp(-jnp.abs(x)))

    o_sc = s[:, :C]
    n_raw = s[:, C:C + C * NNEG]
    non_pad = (ow != PAD).astype(jnp.float32)
    n_valid = jnp.sum(non_pad)
    oloss = jnp.sum(log_sigmoid(o_sc) * non_pad) / n_valid
    nterm = jnp.sum(log_sigmoid(-n_raw)) / (C * B)
    out_ref[0, 0] = -(oloss + nterm)


def _tc_loss(scores, owords):
    return pl.pallas_call(
        _tc_loss_body,
        out_shape=jax.ShapeDtypeStruct((1, 1), jnp.float32),
        in_specs=[
            pl.BlockSpec(memory_space=pltpu.VMEM),
            pl.BlockSpec(memory_space=pltpu.VMEM),
        ],
        out_specs=pl.BlockSpec(memory_space=pltpu.SMEM),
    )(scores, owords)


def kernel(iword, owords, nwords, ivec_table, ovec_table):
    pad = jnp.zeros((B, KP - K), jnp.int32)
    okidx = jnp.concatenate([owords, nwords, pad], axis=1)
    scores = _sc_scores(iword, okidx, ovec_table, ivec_table)
    loss = _tc_loss(scores, owords)
    return loss[0, 0]
